# baseline (speedup 1.0000x reference)
import jax
import jax.numpy as jnp
from jax.experimental import pallas as pl
from jax.experimental.pallas import tpu as pltpu

ROWS = 16384
COLS = 100
NUM_BINS = 100
BLK = 2048


def _body(loss_ref, labels_ref, bw_ref, out_ref):
    ones = jnp.ones((COLS, 128), jnp.float32)
    slab = jax.lax.dot_general(
        labels_ref[...], ones, (((1,), (0,)), ((), ())),
        preferred_element_type=jnp.float32)
    m = slab / COLS
    idx = jnp.clip((m * NUM_BINS).astype(jnp.int32), 0, NUM_BINS - 1)
    bw_b = jnp.broadcast_to(bw_ref[...], (BLK, 128))
    w = jnp.take_along_axis(bw_b, idx, axis=1)
    partial = jnp.sum(w[:, :COLS] * loss_ref[...]).reshape(1, 1)

    @pl.when(pl.program_id(0) == 0)
    def _():
        out_ref[...] = jnp.zeros((1, 1), jnp.float32)

    out_ref[...] += partial


def kernel(loss, labels, bin_weights):
    bw_pad = jnp.pad(bin_weights, (0, 128 - NUM_BINS)).reshape(1, 128)
    grid = (ROWS // BLK,)
    out = pl.pallas_call(
        _body,
        grid=grid,
        in_specs=[
            pl.BlockSpec((BLK, COLS), lambda i: (i, 0)),
            pl.BlockSpec((BLK, COLS), lambda i: (i, 0)),
            pl.BlockSpec((1, 128), lambda i: (0, 0)),
        ],
        out_specs=pl.BlockSpec((1, 1), lambda i: (0, 0)),
        out_shape=jax.ShapeDtypeStruct((1, 1), jnp.float32),
    )(loss, labels, bw_pad)
    return out[0, 0] * (1.0 / (ROWS * COLS))


# speedup vs baseline: 1.0597x; 1.0597x over previous
import jax
import jax.numpy as jnp
from jax.experimental import pallas as pl
from jax.experimental.pallas import tpu as pltpu

ROWS = 16384
COLS = 100
NUM_BINS = 100
BLK = 4096


def _body(loss_ref, labels_ref, bw_ref, out_ref):
    ones = jnp.ones((COLS, 128), jnp.float32)
    slab = jax.lax.dot_general(
        labels_ref[...], ones, (((1,), (0,)), ((), ())),
        preferred_element_type=jnp.float32)
    m = slab / COLS
    idx = jnp.clip((m * NUM_BINS).astype(jnp.int32), 0, NUM_BINS - 1)
    bw_b = jnp.broadcast_to(bw_ref[...], (BLK, 128))
    w = jnp.take_along_axis(bw_b, idx, axis=1)
    partial = jnp.sum(w[:, :COLS] * loss_ref[...]).reshape(1, 1)

    @pl.when(pl.program_id(0) == 0)
    def _():
        out_ref[...] = jnp.zeros((1, 1), jnp.float32)

    out_ref[...] += partial


def kernel(loss, labels, bin_weights):
    bw_pad = jnp.pad(bin_weights, (0, 128 - NUM_BINS)).reshape(1, 128)
    grid = (ROWS // BLK,)
    out = pl.pallas_call(
        _body,
        grid=grid,
        in_specs=[
            pl.BlockSpec((BLK, COLS), lambda i: (i, 0)),
            pl.BlockSpec((BLK, COLS), lambda i: (i, 0)),
            pl.BlockSpec((1, 128), lambda i: (0, 0)),
        ],
        out_specs=pl.BlockSpec((1, 1), lambda i: (0, 0)),
        out_shape=jax.ShapeDtypeStruct((1, 1), jnp.float32),
    )(loss, labels, bw_pad)
    return out[0, 0] * (1.0 / (ROWS * COLS))
